# initial kernel scaffold (unmeasured)
import jax
import jax.numpy as jnp
from jax import lax
from jax.experimental import pallas as pl
from jax.experimental.pallas import tpu as pltpu

N_DEV = 8


def kernel(x, Win0, Wout0, Win1, Wout1, Win2, Wout2):
    b, d = x.shape
    B = N_DEV * b

    def body(x_ref, win0_ref, wout0_ref, win1_ref, wout1_ref,
             win2_ref, wout2_ref, out_ref,
             xfull, pacc, rs_send, rs_recv,
             ag_send_sems, ag_recv_sems, rs_send_sems, rs_recv_sems):
        my = lax.axis_index("i")
        left = lax.rem(my + N_DEV - 1, N_DEV)
        right = lax.rem(my + 1, N_DEV)

        barrier_sem = pltpu.get_barrier_semaphore()
        for nbr in (left, right):
            pl.semaphore_signal(
                barrier_sem, inc=1,
                device_id=(nbr,), device_id_type=pl.DeviceIdType.MESH,
            )
        pl.semaphore_wait(barrier_sem, 2)

        xfull[pl.ds(my * b, b), :] = x_ref[...].astype(jnp.bfloat16)

        weight_refs = [(win0_ref, wout0_ref),
                       (win1_ref, wout1_ref),
                       (win2_ref, wout2_ref)]

        for l in range(3):
            win_ref, wout_ref = weight_refs[l]

            for h in range(N_DEV - 1):
                origin = lax.rem(my - h + N_DEV, N_DEV)
                rdma = pltpu.make_async_remote_copy(
                    src_ref=xfull.at[pl.ds(origin * b, b), :],
                    dst_ref=xfull.at[pl.ds(origin * b, b), :],
                    send_sem=ag_send_sems.at[h],
                    recv_sem=ag_recv_sems.at[h],
                    device_id=(right,),
                    device_id_type=pl.DeviceIdType.MESH,
                )
                rdma.start()
                rdma.wait()

            h_pre = jnp.dot(xfull[...], win_ref[...].astype(jnp.bfloat16),
                            preferred_element_type=jnp.float32)
            h_act = jnp.maximum(h_pre, 0.0).astype(jnp.bfloat16)
            pacc[...] = jnp.dot(h_act, wout_ref[...].astype(jnp.bfloat16),
                                preferred_element_type=jnp.float32)

            for s in range(N_DEV - 1):
                c = lax.rem(my - s - 1 + N_DEV, N_DEV)
                contrib = pacc[pl.ds(c * b, b), :]
                if s == 0:
                    rs_send[s] = contrib
                else:
                    rs_send[s] = contrib + rs_recv[s - 1]
                rdma = pltpu.make_async_remote_copy(
                    src_ref=rs_send.at[s],
                    dst_ref=rs_recv.at[s],
                    send_sem=rs_send_sems.at[s],
                    recv_sem=rs_recv_sems.at[s],
                    device_id=(right,),
                    device_id_type=pl.DeviceIdType.MESH,
                )
                rdma.start()
                rdma.wait()

            res = pacc[pl.ds(my * b, b), :] + rs_recv[N_DEV - 2]
            if l < 2:
                xfull[pl.ds(my * b, b), :] = res.astype(jnp.bfloat16)
            else:
                out_ref[...] = res

    return pl.pallas_call(
        body,
        out_shape=jax.ShapeDtypeStruct((b, d), jnp.float32),
        in_specs=[pl.BlockSpec(memory_space=pltpu.VMEM)] * 7,
        out_specs=pl.BlockSpec(memory_space=pltpu.VMEM),
        scratch_shapes=[
            pltpu.VMEM((B, d), jnp.bfloat16),
            pltpu.VMEM((B, d), jnp.float32),
            pltpu.VMEM((N_DEV - 1, b, d), jnp.float32),
            pltpu.VMEM((N_DEV - 1, b, d), jnp.float32),
            pltpu.SemaphoreType.DMA((N_DEV - 1,)),
            pltpu.SemaphoreType.DMA((N_DEV - 1,)),
            pltpu.SemaphoreType.DMA((N_DEV - 1,)),
            pltpu.SemaphoreType.DMA((N_DEV - 1,)),
        ],
        compiler_params=pltpu.CompilerParams(collective_id=0),
    )(x, Win0, Wout0, Win1, Wout1, Win2, Wout2)


# baseline (device time: 217862 ns/iter reference)
import jax
import jax.numpy as jnp
from jax import lax
from jax.experimental import pallas as pl
from jax.experimental.pallas import tpu as pltpu

N_DEV = 8


def kernel(x, Win0, Wout0, Win1, Wout1, Win2, Wout2):
    b, d = x.shape
    B = N_DEV * b

    def body(x_ref, win0_ref, wout0_ref, win1_ref, wout1_ref,
             win2_ref, wout2_ref, out_ref,
             xfull, pacc, rs_send, rs_recv,
             ag_send_sems, ag_recv_sems, rs_send_sems, rs_recv_sems):
        my = lax.axis_index("i")
        left = lax.rem(my + N_DEV - 1, N_DEV)
        right = lax.rem(my + 1, N_DEV)

        barrier_sem = pltpu.get_barrier_semaphore()
        for nbr in (left, right):
            pl.semaphore_signal(
                barrier_sem, inc=1,
                device_id=(nbr,), device_id_type=pl.DeviceIdType.MESH,
            )
        pl.semaphore_wait(barrier_sem, 2)

        xfull[pl.ds(my * b, b), :] = x_ref[...].astype(jnp.bfloat16)

        weight_refs = [(win0_ref, wout0_ref),
                       (win1_ref, wout1_ref),
                       (win2_ref, wout2_ref)]

        for l in range(3):
            win_ref, wout_ref = weight_refs[l]

            for h in range(N_DEV - 1):
                origin = lax.rem(my - h + N_DEV, N_DEV)
                rdma = pltpu.make_async_remote_copy(
                    src_ref=xfull.at[pl.ds(origin * b, b), :],
                    dst_ref=xfull.at[pl.ds(origin * b, b), :],
                    send_sem=ag_send_sems.at[h],
                    recv_sem=ag_recv_sems.at[h],
                    device_id=(right,),
                    device_id_type=pl.DeviceIdType.MESH,
                )
                rdma.start()
                rdma.wait()

            h_pre = jnp.dot(xfull[...], win_ref[...],
                            preferred_element_type=jnp.float32)
            h_act = jnp.maximum(h_pre, 0.0).astype(jnp.bfloat16)
            pacc[...] = jnp.dot(h_act, wout_ref[...],
                                preferred_element_type=jnp.float32)

            for s in range(N_DEV - 1):
                c = lax.rem(my - s - 1 + N_DEV, N_DEV)
                contrib = pacc[pl.ds(c * b, b), :]
                if s == 0:
                    rs_send[s] = contrib
                else:
                    rs_send[s] = contrib + rs_recv[s - 1]
                rdma = pltpu.make_async_remote_copy(
                    src_ref=rs_send.at[s],
                    dst_ref=rs_recv.at[s],
                    send_sem=rs_send_sems.at[s],
                    recv_sem=rs_recv_sems.at[s],
                    device_id=(right,),
                    device_id_type=pl.DeviceIdType.MESH,
                )
                rdma.start()
                rdma.wait()

            res = pacc[pl.ds(my * b, b), :] + rs_recv[N_DEV - 2]
            if l < 2:
                xfull[pl.ds(my * b, b), :] = res.astype(jnp.bfloat16)
            else:
                out_ref[...] = res

    return pl.pallas_call(
        body,
        out_shape=jax.ShapeDtypeStruct((b, d), jnp.float32),
        in_specs=[pl.BlockSpec(memory_space=pltpu.VMEM)] * 7,
        out_specs=pl.BlockSpec(memory_space=pltpu.VMEM),
        scratch_shapes=[
            pltpu.VMEM((B, d), jnp.bfloat16),
            pltpu.VMEM((B, d), jnp.float32),
            pltpu.VMEM((N_DEV - 1, b, d), jnp.float32),
            pltpu.VMEM((N_DEV - 1, b, d), jnp.float32),
            pltpu.SemaphoreType.DMA((N_DEV - 1,)),
            pltpu.SemaphoreType.DMA((N_DEV - 1,)),
            pltpu.SemaphoreType.DMA((N_DEV - 1,)),
            pltpu.SemaphoreType.DMA((N_DEV - 1,)),
        ],
        compiler_params=pltpu.CompilerParams(
            collective_id=0,
            vmem_limit_bytes=100 * 1024 * 1024,
        ),
    )(x,
      Win0.astype(jnp.bfloat16), Wout0.astype(jnp.bfloat16),
      Win1.astype(jnp.bfloat16), Wout1.astype(jnp.bfloat16),
      Win2.astype(jnp.bfloat16), Wout2.astype(jnp.bfloat16))


# device time: 105474 ns/iter; 2.0656x vs baseline; 2.0656x over previous
import jax
import jax.numpy as jnp
from jax import lax
from jax.experimental import pallas as pl
from jax.experimental.pallas import tpu as pltpu

N_DEV = 8


def kernel(x, Win0, Wout0, Win1, Wout1, Win2, Wout2):
    b, d = x.shape
    B = N_DEV * b

    def body(x_ref, win0_ref, wout0_ref, win1_ref, wout1_ref,
             win2_ref, wout2_ref, out_ref,
             xfull, pacc, rs_send, rs_recv,
             ag_send_sems, ag_recv_sems, rs_send_sems, rs_recv_sems):
        my = lax.axis_index("i")

        def others():
            return [lax.rem(my + o, N_DEV) for o in range(1, N_DEV)]

        barrier_sem = pltpu.get_barrier_semaphore()
        for t in others():
            pl.semaphore_signal(
                barrier_sem, inc=1,
                device_id=(t,), device_id_type=pl.DeviceIdType.MESH,
            )
        pl.semaphore_wait(barrier_sem, N_DEV - 1)

        xfull[pl.ds(my * b, b), :] = x_ref[...].astype(jnp.bfloat16)

        weight_refs = [(win0_ref, wout0_ref),
                       (win1_ref, wout1_ref),
                       (win2_ref, wout2_ref)]

        my_chunk = pl.ds(my * b, b)

        for l in range(3):
            win_ref, wout_ref = weight_refs[l]

            ag_sends = []
            for t in others():
                rdma = pltpu.make_async_remote_copy(
                    src_ref=xfull.at[my_chunk, :],
                    dst_ref=xfull.at[my_chunk, :],
                    send_sem=ag_send_sems.at[t],
                    recv_sem=ag_recv_sems.at[my],
                    device_id=(t,),
                    device_id_type=pl.DeviceIdType.MESH,
                )
                rdma.start()
                ag_sends.append(rdma)

            for k in others():
                recv = pltpu.make_async_remote_copy(
                    src_ref=xfull.at[my_chunk, :],
                    dst_ref=xfull.at[pl.ds(k * b, b), :],
                    send_sem=ag_send_sems.at[k],
                    recv_sem=ag_recv_sems.at[k],
                    device_id=(k,),
                    device_id_type=pl.DeviceIdType.MESH,
                )
                recv.wait_recv()
            for rdma in ag_sends:
                rdma.wait_send()

            h_pre = jnp.dot(xfull[...], win_ref[...],
                            preferred_element_type=jnp.float32)
            h_act = jnp.maximum(h_pre, 0.0).astype(jnp.bfloat16)
            pacc[...] = jnp.dot(h_act, wout_ref[...],
                                preferred_element_type=jnp.float32)
            rs_send[...] = pacc[...].astype(jnp.bfloat16).reshape(
                N_DEV, b, d)

            rs_sends = []
            for t in others():
                rdma = pltpu.make_async_remote_copy(
                    src_ref=rs_send.at[t],
                    dst_ref=rs_recv.at[my],
                    send_sem=rs_send_sems.at[t],
                    recv_sem=rs_recv_sems.at[my],
                    device_id=(t,),
                    device_id_type=pl.DeviceIdType.MESH,
                )
                rdma.start()
                rs_sends.append(rdma)

            for k in others():
                recv = pltpu.make_async_remote_copy(
                    src_ref=rs_send.at[k],
                    dst_ref=rs_recv.at[k],
                    send_sem=rs_send_sems.at[k],
                    recv_sem=rs_recv_sems.at[k],
                    device_id=(k,),
                    device_id_type=pl.DeviceIdType.MESH,
                )
                recv.wait_recv()
            for rdma in rs_sends:
                rdma.wait_send()

            res = pacc[my_chunk, :]
            for k in others():
                res = res + rs_recv[k].astype(jnp.float32)

            if l < 2:
                xfull[my_chunk, :] = res.astype(jnp.bfloat16)
            else:
                out_ref[...] = res

    return pl.pallas_call(
        body,
        out_shape=jax.ShapeDtypeStruct((b, d), jnp.float32),
        in_specs=[pl.BlockSpec(memory_space=pltpu.VMEM)] * 7,
        out_specs=pl.BlockSpec(memory_space=pltpu.VMEM),
        scratch_shapes=[
            pltpu.VMEM((B, d), jnp.bfloat16),
            pltpu.VMEM((B, d), jnp.float32),
            pltpu.VMEM((N_DEV, b, d), jnp.bfloat16),
            pltpu.VMEM((N_DEV, b, d), jnp.bfloat16),
            pltpu.SemaphoreType.DMA((N_DEV,)),
            pltpu.SemaphoreType.DMA((N_DEV,)),
            pltpu.SemaphoreType.DMA((N_DEV,)),
            pltpu.SemaphoreType.DMA((N_DEV,)),
        ],
        compiler_params=pltpu.CompilerParams(
            collective_id=0,
            vmem_limit_bytes=100 * 1024 * 1024,
        ),
    )(x,
      Win0.astype(jnp.bfloat16), Wout0.astype(jnp.bfloat16),
      Win1.astype(jnp.bfloat16), Wout1.astype(jnp.bfloat16),
      Win2.astype(jnp.bfloat16), Wout2.astype(jnp.bfloat16))


# device time: 104217 ns/iter; 2.0905x vs baseline; 1.0121x over previous
import jax
import jax.numpy as jnp
from jax import lax
from jax.experimental import pallas as pl
from jax.experimental.pallas import tpu as pltpu

N_DEV = 8


def kernel(x, Win0, Wout0, Win1, Wout1, Win2, Wout2):
    b, d = x.shape
    B = N_DEV * b

    def body(x_ref, win0_ref, wout0_ref, win1_ref, wout1_ref,
             win2_ref, wout2_ref, out_ref,
             xfull, h_act, rs_send, rs_recv,
             ag_send_sems, ag_recv_sems, rs_send_sems, rs_recv_sems):
        my = lax.axis_index("i")

        def peer(o):
            return lax.rem(my + o, N_DEV)

        barrier_sem = pltpu.get_barrier_semaphore()
        for o in range(1, N_DEV):
            pl.semaphore_signal(
                barrier_sem, inc=1,
                device_id=(peer(o),), device_id_type=pl.DeviceIdType.MESH,
            )
        pl.semaphore_wait(barrier_sem, N_DEV - 1)

        my_chunk = pl.ds(my * b, b)
        xfull[my_chunk, :] = x_ref[...].astype(jnp.bfloat16)

        weight_refs = [(win0_ref, wout0_ref),
                       (win1_ref, wout1_ref),
                       (win2_ref, wout2_ref)]

        for l in range(3):
            win_ref, wout_ref = weight_refs[l]

            ag_sends = []
            for o in range(1, N_DEV):
                t = peer(o)
                rdma = pltpu.make_async_remote_copy(
                    src_ref=xfull.at[my_chunk, :],
                    dst_ref=xfull.at[my_chunk, :],
                    send_sem=ag_send_sems.at[t],
                    recv_sem=ag_recv_sems.at[my],
                    device_id=(t,),
                    device_id_type=pl.DeviceIdType.MESH,
                )
                rdma.start()
                ag_sends.append(rdma)

            def wait_chunk(k):
                recv = pltpu.make_async_remote_copy(
                    src_ref=xfull.at[my_chunk, :],
                    dst_ref=xfull.at[pl.ds(k * b, b), :],
                    send_sem=ag_send_sems.at[k],
                    recv_sem=ag_recv_sems.at[k],
                    device_id=(k,),
                    device_id_type=pl.DeviceIdType.MESH,
                )
                recv.wait_recv()

            def gemm1_pair(k1, k2):
                blk = jnp.concatenate(
                    [xfull[pl.ds(k1 * b, b), :], xfull[pl.ds(k2 * b, b), :]],
                    axis=0)
                hp = jnp.dot(blk, win_ref[...],
                             preferred_element_type=jnp.float32)
                hb = jnp.maximum(hp, 0.0).astype(jnp.bfloat16)
                h_act[pl.ds(k1 * b, b), :] = hb[:b, :]
                h_act[pl.ds(k2 * b, b), :] = hb[b:, :]

            wait_chunk(peer(1))
            gemm1_pair(my, peer(1))
            for o in (2, 4, 6):
                wait_chunk(peer(o))
                wait_chunk(peer(o + 1))
                gemm1_pair(peer(o), peer(o + 1))

            rs_sends = []
            own_val = None
            pair_list = [(peer(1), peer(2), True, True),
                         (peer(3), peer(4), True, True),
                         (peer(5), peer(6), True, True),
                         (peer(7), my, True, False)]
            for c1, c2, send1, send2 in pair_list:
                blk = jnp.concatenate(
                    [h_act[pl.ds(c1 * b, b), :], h_act[pl.ds(c2 * b, b), :]],
                    axis=0)
                pb = jnp.dot(blk, wout_ref[...],
                             preferred_element_type=jnp.float32)
                for idx, c, send in ((0, c1, send1), (1, c2, send2)):
                    sub = pb[idx * b:(idx + 1) * b, :]
                    if not send:
                        own_val = sub
                        continue
                    rs_send[pl.ds(c * b, b), :] = sub.astype(jnp.bfloat16)
                    rdma = pltpu.make_async_remote_copy(
                        src_ref=rs_send.at[pl.ds(c * b, b), :],
                        dst_ref=rs_recv.at[pl.ds(my * b, b), :],
                        send_sem=rs_send_sems.at[c],
                        recv_sem=rs_recv_sems.at[my],
                        device_id=(c,),
                        device_id_type=pl.DeviceIdType.MESH,
                    )
                    rdma.start()
                    rs_sends.append(rdma)

            res = own_val
            for o in range(1, N_DEV):
                k = peer(o)
                recv = pltpu.make_async_remote_copy(
                    src_ref=rs_send.at[pl.ds(k * b, b), :],
                    dst_ref=rs_recv.at[pl.ds(k * b, b), :],
                    send_sem=rs_send_sems.at[k],
                    recv_sem=rs_recv_sems.at[k],
                    device_id=(k,),
                    device_id_type=pl.DeviceIdType.MESH,
                )
                recv.wait_recv()
                res = res + rs_recv[pl.ds(k * b, b), :].astype(jnp.float32)

            for rdma in ag_sends:
                rdma.wait_send()
            for rdma in rs_sends:
                rdma.wait_send()

            if l < 2:
                xfull[my_chunk, :] = res.astype(jnp.bfloat16)
            else:
                out_ref[...] = res

    return pl.pallas_call(
        body,
        out_shape=jax.ShapeDtypeStruct((b, d), jnp.float32),
        in_specs=[pl.BlockSpec(memory_space=pltpu.VMEM)] * 7,
        out_specs=pl.BlockSpec(memory_space=pltpu.VMEM),
        scratch_shapes=[
            pltpu.VMEM((B, d), jnp.bfloat16),
            pltpu.VMEM((B, 2 * d), jnp.bfloat16),
            pltpu.VMEM((B, d), jnp.bfloat16),
            pltpu.VMEM((B, d), jnp.bfloat16),
            pltpu.SemaphoreType.DMA((N_DEV,)),
            pltpu.SemaphoreType.DMA((N_DEV,)),
            pltpu.SemaphoreType.DMA((N_DEV,)),
            pltpu.SemaphoreType.DMA((N_DEV,)),
        ],
        compiler_params=pltpu.CompilerParams(
            collective_id=0,
            vmem_limit_bytes=100 * 1024 * 1024,
        ),
    )(x,
      Win0.astype(jnp.bfloat16), Wout0.astype(jnp.bfloat16),
      Win1.astype(jnp.bfloat16), Wout1.astype(jnp.bfloat16),
      Win2.astype(jnp.bfloat16), Wout2.astype(jnp.bfloat16))


# device time: 71496 ns/iter; 3.0472x vs baseline; 1.4577x over previous
import jax
import jax.numpy as jnp
from jax import lax
from jax.experimental import pallas as pl
from jax.experimental.pallas import tpu as pltpu

N_DEV = 8


def kernel(x, Win0, Wout0, Win1, Wout1, Win2, Wout2):
    b, d = x.shape
    B = N_DEV * b

    def body(x_ref, win0_ref, wout0_ref, win1_ref, wout1_ref,
             win2_ref, wout2_ref, out_ref,
             xfull, h_act, rs_send, rs_recv,
             win_f32, wout_f32, win_bf, wout_bf,
             ag_send_sems, ag_recv_sems, rs_send_sems, rs_recv_sems,
             w_sems):
        my = lax.axis_index("i")

        def peer(o):
            return lax.rem(my + o, N_DEV)

        barrier_sem = pltpu.get_barrier_semaphore()
        for o in range(1, N_DEV):
            pl.semaphore_signal(
                barrier_sem, inc=1,
                device_id=(peer(o),), device_id_type=pl.DeviceIdType.MESH,
            )
        pl.semaphore_wait(barrier_sem, N_DEV - 1)

        my_chunk = pl.ds(my * b, b)
        xfull[my_chunk, :] = x_ref[...].astype(jnp.bfloat16)

        weight_refs = [(win0_ref, wout0_ref),
                       (win1_ref, wout1_ref),
                       (win2_ref, wout2_ref)]

        for l in range(3):
            win_ref, wout_ref = weight_refs[l]

            win_dma = pltpu.make_async_copy(win_ref, win_f32, w_sems.at[0])
            wout_dma = pltpu.make_async_copy(wout_ref, wout_f32, w_sems.at[1])
            win_dma.start()
            wout_dma.start()

            ag_sends = []
            for o in range(1, N_DEV):
                t = peer(o)
                rdma = pltpu.make_async_remote_copy(
                    src_ref=xfull.at[my_chunk, :],
                    dst_ref=xfull.at[my_chunk, :],
                    send_sem=ag_send_sems.at[t],
                    recv_sem=ag_recv_sems.at[my],
                    device_id=(t,),
                    device_id_type=pl.DeviceIdType.MESH,
                )
                rdma.start()
                ag_sends.append(rdma)

            win_dma.wait()
            win_bf[...] = win_f32[...].astype(jnp.bfloat16)
            wout_dma.wait()
            wout_bf[...] = wout_f32[...].astype(jnp.bfloat16)

            def wait_chunk(k):
                recv = pltpu.make_async_remote_copy(
                    src_ref=xfull.at[my_chunk, :],
                    dst_ref=xfull.at[pl.ds(k * b, b), :],
                    send_sem=ag_send_sems.at[k],
                    recv_sem=ag_recv_sems.at[k],
                    device_id=(k,),
                    device_id_type=pl.DeviceIdType.MESH,
                )
                recv.wait_recv()

            def gemm1_pair(k1, k2):
                blk = jnp.concatenate(
                    [xfull[pl.ds(k1 * b, b), :], xfull[pl.ds(k2 * b, b), :]],
                    axis=0)
                hp = jnp.dot(blk, win_bf[...],
                             preferred_element_type=jnp.float32)
                hb = jnp.maximum(hp, 0.0).astype(jnp.bfloat16)
                h_act[pl.ds(k1 * b, b), :] = hb[:b, :]
                h_act[pl.ds(k2 * b, b), :] = hb[b:, :]

            wait_chunk(peer(1))
            gemm1_pair(my, peer(1))
            for o in (2, 4, 6):
                wait_chunk(peer(o))
                wait_chunk(peer(o + 1))
                gemm1_pair(peer(o), peer(o + 1))

            rs_sends = []
            own_val = None
            pair_list = [(peer(1), peer(2), True, True),
                         (peer(3), peer(4), True, True),
                         (peer(5), peer(6), True, True),
                         (peer(7), my, True, False)]
            for c1, c2, send1, send2 in pair_list:
                blk = jnp.concatenate(
                    [h_act[pl.ds(c1 * b, b), :], h_act[pl.ds(c2 * b, b), :]],
                    axis=0)
                pb = jnp.dot(blk, wout_bf[...],
                             preferred_element_type=jnp.float32)
                for idx, c, send in ((0, c1, send1), (1, c2, send2)):
                    sub = pb[idx * b:(idx + 1) * b, :]
                    if not send:
                        own_val = sub
                        continue
                    rs_send[pl.ds(c * b, b), :] = sub.astype(jnp.bfloat16)
                    rdma = pltpu.make_async_remote_copy(
                        src_ref=rs_send.at[pl.ds(c * b, b), :],
                        dst_ref=rs_recv.at[pl.ds(my * b, b), :],
                        send_sem=rs_send_sems.at[c],
                        recv_sem=rs_recv_sems.at[my],
                        device_id=(c,),
                        device_id_type=pl.DeviceIdType.MESH,
                    )
                    rdma.start()
                    rs_sends.append(rdma)

            res = own_val
            for o in range(1, N_DEV):
                k = peer(o)
                recv = pltpu.make_async_remote_copy(
                    src_ref=rs_send.at[pl.ds(k * b, b), :],
                    dst_ref=rs_recv.at[pl.ds(k * b, b), :],
                    send_sem=rs_send_sems.at[k],
                    recv_sem=rs_recv_sems.at[k],
                    device_id=(k,),
                    device_id_type=pl.DeviceIdType.MESH,
                )
                recv.wait_recv()
                res = res + rs_recv[pl.ds(k * b, b), :].astype(jnp.float32)

            for rdma in ag_sends:
                rdma.wait_send()
            for rdma in rs_sends:
                rdma.wait_send()

            if l < 2:
                xfull[my_chunk, :] = res.astype(jnp.bfloat16)
            else:
                out_ref[...] = res

    return pl.pallas_call(
        body,
        out_shape=jax.ShapeDtypeStruct((b, d), jnp.float32),
        in_specs=[pl.BlockSpec(memory_space=pltpu.VMEM)]
        + [pl.BlockSpec(memory_space=pl.MemorySpace.ANY)] * 6,
        out_specs=pl.BlockSpec(memory_space=pltpu.VMEM),
        scratch_shapes=[
            pltpu.VMEM((B, d), jnp.bfloat16),
            pltpu.VMEM((B, 2 * d), jnp.bfloat16),
            pltpu.VMEM((B, d), jnp.bfloat16),
            pltpu.VMEM((B, d), jnp.bfloat16),
            pltpu.VMEM(Win0.shape, jnp.float32),
            pltpu.VMEM(Wout0.shape, jnp.float32),
            pltpu.VMEM(Win0.shape, jnp.bfloat16),
            pltpu.VMEM(Wout0.shape, jnp.bfloat16),
            pltpu.SemaphoreType.DMA((N_DEV,)),
            pltpu.SemaphoreType.DMA((N_DEV,)),
            pltpu.SemaphoreType.DMA((N_DEV,)),
            pltpu.SemaphoreType.DMA((N_DEV,)),
            pltpu.SemaphoreType.DMA((2,)),
        ],
        compiler_params=pltpu.CompilerParams(
            collective_id=0,
            vmem_limit_bytes=100 * 1024 * 1024,
        ),
    )(x, Win0, Wout0, Win1, Wout1, Win2, Wout2)


# device time: 65434 ns/iter; 3.3295x vs baseline; 1.0926x over previous
import jax
import jax.numpy as jnp
from jax import lax
from jax.experimental import pallas as pl
from jax.experimental.pallas import tpu as pltpu

N_DEV = 8


def kernel(x, Win0, Wout0, Win1, Wout1, Win2, Wout2):
    b, d = x.shape
    B = N_DEV * b

    def body(x_ref, win0_ref, wout0_ref, win1_ref, wout1_ref,
             win2_ref, wout2_ref, out_ref,
             xfull, rs_send, rs_recv,
             win_f32, wout_f32, win_bf, wout_bf,
             ag_send_sems, ag_recv_sems, rs_send_sems, rs_recv_sems,
             w_sems):
        my = lax.axis_index("i")

        def peer(o):
            return lax.rem(my + o, N_DEV)

        barrier_sem = pltpu.get_barrier_semaphore()
        for o in range(1, N_DEV):
            pl.semaphore_signal(
                barrier_sem, inc=1,
                device_id=(peer(o),), device_id_type=pl.DeviceIdType.MESH,
            )
        pl.semaphore_wait(barrier_sem, N_DEV - 1)

        my_chunk = pl.ds(my * b, b)
        xfull[my_chunk, :] = x_ref[...].astype(jnp.bfloat16)

        weight_refs = [(win0_ref, wout0_ref),
                       (win1_ref, wout1_ref),
                       (win2_ref, wout2_ref)]

        for l in range(3):
            win_ref, wout_ref = weight_refs[l]

            win_dma = pltpu.make_async_copy(win_ref, win_f32, w_sems.at[0])
            wout_dma = pltpu.make_async_copy(wout_ref, wout_f32, w_sems.at[1])
            win_dma.start()
            wout_dma.start()

            ag_sends = []
            for o in range(1, N_DEV):
                t = peer(o)
                rdma = pltpu.make_async_remote_copy(
                    src_ref=xfull.at[my_chunk, :],
                    dst_ref=xfull.at[my_chunk, :],
                    send_sem=ag_send_sems.at[t],
                    recv_sem=ag_recv_sems.at[my],
                    device_id=(t,),
                    device_id_type=pl.DeviceIdType.MESH,
                )
                rdma.start()
                ag_sends.append(rdma)

            win_dma.wait()
            win_bf[...] = win_f32[...].astype(jnp.bfloat16)
            wout_dma.wait()
            wout_bf[...] = wout_f32[...].astype(jnp.bfloat16)

            def wait_chunk(k):
                recv = pltpu.make_async_remote_copy(
                    src_ref=xfull.at[my_chunk, :],
                    dst_ref=xfull.at[pl.ds(k * b, b), :],
                    send_sem=ag_send_sems.at[k],
                    recv_sem=ag_recv_sems.at[k],
                    device_id=(k,),
                    device_id_type=pl.DeviceIdType.MESH,
                )
                recv.wait_recv()

            rs_sends = []
            own_val = None
            for o in (0, 2, 4, 6):
                k1, k2 = peer(o), peer(o + 1)
                if o == 0:
                    wait_chunk(k2)
                else:
                    wait_chunk(k1)
                    wait_chunk(k2)
                blk = jnp.concatenate(
                    [xfull[pl.ds(k1 * b, b), :], xfull[pl.ds(k2 * b, b), :]],
                    axis=0)
                hp = jnp.dot(blk, win_bf[...],
                             preferred_element_type=jnp.float32)
                hb = jnp.maximum(hp, 0.0).astype(jnp.bfloat16)
                pb = jnp.dot(hb, wout_bf[...],
                             preferred_element_type=jnp.float32)
                for idx, c in ((0, k1), (1, k2)):
                    sub = pb[idx * b:(idx + 1) * b, :]
                    if o == 0 and idx == 0:
                        own_val = sub
                        continue
                    rs_send[pl.ds(c * b, b), :] = sub.astype(jnp.bfloat16)
                    rdma = pltpu.make_async_remote_copy(
                        src_ref=rs_send.at[pl.ds(c * b, b), :],
                        dst_ref=rs_recv.at[pl.ds(my * b, b), :],
                        send_sem=rs_send_sems.at[c],
                        recv_sem=rs_recv_sems.at[my],
                        device_id=(c,),
                        device_id_type=pl.DeviceIdType.MESH,
                    )
                    rdma.start()
                    rs_sends.append(rdma)

            res = own_val
            for o in range(1, N_DEV):
                k = peer(o)
                recv = pltpu.make_async_remote_copy(
                    src_ref=rs_send.at[pl.ds(k * b, b), :],
                    dst_ref=rs_recv.at[pl.ds(k * b, b), :],
                    send_sem=rs_send_sems.at[k],
                    recv_sem=rs_recv_sems.at[k],
                    device_id=(k,),
                    device_id_type=pl.DeviceIdType.MESH,
                )
                recv.wait_recv()
                res = res + rs_recv[pl.ds(k * b, b), :].astype(jnp.float32)

            for rdma in ag_sends:
                rdma.wait_send()
            for rdma in rs_sends:
                rdma.wait_send()

            if l < 2:
                xfull[my_chunk, :] = res.astype(jnp.bfloat16)
            else:
                out_ref[...] = res

    return pl.pallas_call(
        body,
        out_shape=jax.ShapeDtypeStruct((b, d), jnp.float32),
        in_specs=[pl.BlockSpec(memory_space=pltpu.VMEM)]
        + [pl.BlockSpec(memory_space=pl.MemorySpace.ANY)] * 6,
        out_specs=pl.BlockSpec(memory_space=pltpu.VMEM),
        scratch_shapes=[
            pltpu.VMEM((B, d), jnp.bfloat16),
            pltpu.VMEM((B, d), jnp.bfloat16),
            pltpu.VMEM((B, d), jnp.bfloat16),
            pltpu.VMEM(Win0.shape, jnp.float32),
            pltpu.VMEM(Wout0.shape, jnp.float32),
            pltpu.VMEM(Win0.shape, jnp.bfloat16),
            pltpu.VMEM(Wout0.shape, jnp.bfloat16),
            pltpu.SemaphoreType.DMA((N_DEV,)),
            pltpu.SemaphoreType.DMA((N_DEV,)),
            pltpu.SemaphoreType.DMA((N_DEV,)),
            pltpu.SemaphoreType.DMA((N_DEV,)),
            pltpu.SemaphoreType.DMA((2,)),
        ],
        compiler_params=pltpu.CompilerParams(
            collective_id=0,
            vmem_limit_bytes=100 * 1024 * 1024,
        ),
    )(x, Win0, Wout0, Win1, Wout1, Win2, Wout2)
